# baseline (speedup 1.0000x reference)
import functools

import jax
import jax.numpy as jnp
from jax import lax
from jax.experimental import pallas as pl
from jax.experimental.pallas import tpu as pltpu
from jax.experimental.pallas import tpu_sc as plsc

NUM_ENTITIES = 1000000
NUM_RELATIONS = 1000
EMBED_DIM = 64
BATCH = 16384

_info = plsc.get_sparse_core_info()
NC, NS, L = _info.num_cores, _info.num_subcores, _info.num_lanes
NW = NC * NS
BPW = BATCH // NW
CHUNK = 128
NCHUNK = BPW // CHUNK
GROUPS = BPW // L

_mesh = plsc.VectorSubcoreMesh(core_axis_name="c", subcore_axis_name="s")


@functools.partial(
    pl.kernel,
    mesh=_mesh,
    out_type=jax.ShapeDtypeStruct((BATCH,), jnp.float32),
    compiler_params=pltpu.CompilerParams(
        needs_layout_passes=False, use_tc_tiling_on_sc=False
    ),
    scratch_types=[
        pltpu.VMEM((NCHUNK, CHUNK), jnp.int32),
        pltpu.VMEM((NCHUNK, CHUNK), jnp.int32),
        pltpu.VMEM((NCHUNK, CHUNK), jnp.int32),
        pltpu.VMEM((BPW, EMBED_DIM), jnp.float32),
        pltpu.VMEM((BPW, EMBED_DIM), jnp.float32),
        pltpu.VMEM((BPW, EMBED_DIM), jnp.float32),
        pltpu.VMEM((BPW,), jnp.float32),
        pltpu.SemaphoreType.DMA,
    ],
)
def _transe_sc(head_hbm, relidx_hbm, tail_hbm, ent_hbm, rel_hbm, out_hbm,
               hidx, ridx, tidx, h_rows, r_rows, t_rows, out_v, sem):
    wid = lax.axis_index("s") * NC + lax.axis_index("c")
    base = wid * BPW

    for j in range(NCHUNK):
        off = base + j * CHUNK
        pltpu.sync_copy(head_hbm.at[pl.ds(off, CHUNK)], hidx.at[j])
        pltpu.sync_copy(relidx_hbm.at[pl.ds(off, CHUNK)], ridx.at[j])
        pltpu.sync_copy(tail_hbm.at[pl.ds(off, CHUNK)], tidx.at[j])

    copies = []
    for j in range(NCHUNK):
        rows_sl = pl.ds(j * CHUNK, CHUNK)
        copies.append(pltpu.async_copy(ent_hbm.at[hidx.at[j]], h_rows.at[rows_sl], sem))
        copies.append(pltpu.async_copy(rel_hbm.at[ridx.at[j]], r_rows.at[rows_sl], sem))
        copies.append(pltpu.async_copy(ent_hbm.at[tidx.at[j]], t_rows.at[rows_sl], sem))
    for cp in copies:
        cp.wait()

    half = jnp.float32(0.5)
    three_half = jnp.float32(1.5)

    def group_body(g, carry):
        rows = jnp.full((L,), g * L, jnp.int32) + lax.iota(jnp.int32, L)
        accs = [jnp.zeros((L,), jnp.float32) for _ in range(4)]
        for c in range(EMBED_DIM):
            cols = jnp.full((L,), c, jnp.int32)
            hv = plsc.load_gather(h_rows, [rows, cols])
            rv = plsc.load_gather(r_rows, [rows, cols])
            tv = plsc.load_gather(t_rows, [rows, cols])
            d = (hv + rv) - tv
            accs[c % 4] = accs[c % 4] + d * d
        acc = (accs[0] + accs[1]) + (accs[2] + accs[3])
        ai = plsc.bitcast(acc, jnp.int32)
        y = plsc.bitcast(jnp.full((L,), 0x5F3759DF, jnp.int32) - (ai >> 1),
                         jnp.float32)
        for _ in range(3):
            y = y * (three_half - half * acc * y * y)
        s = acc * y
        out_v[pl.ds(pl.multiple_of(g * L, L), L)] = -s
        return carry

    lax.fori_loop(0, GROUPS, group_body, 0)
    pltpu.sync_copy(out_v, out_hbm.at[pl.ds(base, BPW)])


def kernel(head, relation, tail, entity_embeddings, relation_embeddings):
    return _transe_sc(head, relation, tail, entity_embeddings,
                      relation_embeddings)
